# baseline (speedup 1.0000x reference)
import functools

import jax
import jax.numpy as jnp
from jax import lax
from jax.experimental import pallas as pl
from jax.experimental.pallas import tpu as pltpu
from jax.experimental.pallas import tpu_sc as plsc

N = 10000
E = 320000
D_IN = 128
H = 128
G = 64

NC = 2
NS = 16
NW = NC * NS
EPW = E // NW
CH = 80
NCH = EPW // CH
NPAD = 10240
RPT = NPAD // NS

R = 1000
NB = N // R


def _mesh():
    return plsc.VectorSubcoreMesh(
        core_axis_name="c", subcore_axis_name="s",
        num_cores=NC, num_subcores=NS)


def _deg_body(dst_hbm, out_hbm, idx_v, hist_v):
    c = lax.axis_index("c")
    s = lax.axis_index("s")
    w = c * NS + s
    zero16 = jnp.zeros((16,), jnp.float32)
    one16 = jnp.ones((16,), jnp.float32)

    def zbody(i, carry):
        hist_v[pl.ds(i * 16, 16)] = zero16
        return carry
    lax.fori_loop(0, N // 16, zbody, 0)

    pltpu.sync_copy(dst_hbm.at[pl.ds(w * EPW, EPW)], idx_v)

    def body(i, carry):
        idx = idx_v[pl.ds(i * 16, 16)]
        plsc.addupdate_scatter(hist_v, [idx], one16)
        return carry
    lax.fori_loop(0, EPW // 16, body, 0)

    pltpu.sync_copy(hist_v, out_hbm.at[w])


def _deg(dst):
    f = functools.partial(
        pl.kernel,
        out_type=jax.ShapeDtypeStruct((NW, N), jnp.float32),
        mesh=_mesh(),
        compiler_params=pltpu.CompilerParams(needs_layout_passes=False),
        scratch_types=[
            pltpu.VMEM((EPW,), jnp.int32),
            pltpu.VMEM((N,), jnp.float32),
        ],
    )(_deg_body)
    return f(dst)


NBUF = 2
NOUT = (NCH + NBUF - 1) // NBUF


def _agg_body(y_hbm, src_hbm, dst3_hbm, out_hbm, sidx, didx,
              r0, r1, acc, g0, g1, s0, s1):
    c = lax.axis_index("c")
    s = lax.axis_index("s")
    w = c * NS + s
    rows = (r0, r1)
    gsem = (g0, g1)
    ssem = (s0, s1)
    zero16 = jnp.zeros((16,), jnp.float32)

    def zr(i, carry):
        def zc(j, carry2):
            r0[i, pl.ds(j * 16, 16)] = zero16
            return carry2
        return lax.fori_loop(0, H // 16, zc, carry)
    lax.fori_loop(0, CH, zr, 0)

    def za(k, carry):
        pltpu.sync_copy(r0, acc.at[pl.ds(s * RPT + k * CH, CH)])
        return carry
    lax.fori_loop(0, RPT // CH, za, 0)

    pltpu.sync_copy(src_hbm.at[pl.ds(w * EPW, EPW)], sidx)
    pltpu.sync_copy(dst3_hbm.at[w], didx)
    plsc.subcore_barrier()

    def outer(g, carry):
        for b in range(NBUF):
            i = g * NBUF + b

            @pl.when(g > 0)
            def _(b=b):
                pltpu.make_async_copy(y_hbm.at[pl.ds(0, CH)], rows[b],
                                      ssem[b]).wait()

            @pl.when(i < NCH)
            def _(b=b, i=i):
                pltpu.async_copy(y_hbm.at[sidx.at[pl.ds(i * CH, CH)]],
                                 rows[b], gsem[b])
        for b in range(NBUF):
            i = g * NBUF + b

            @pl.when(i < NCH)
            def _(b=b, i=i):
                pltpu.make_async_copy(y_hbm.at[pl.ds(0, CH)], rows[b],
                                      gsem[b]).wait()
                pltpu.async_copy(rows[b], acc.at[didx.at[i]], ssem[b],
                                 add=True)
        return carry
    lax.fori_loop(0, NOUT, outer, 0)

    for b in range(NBUF):
        if (NOUT - 1) * NBUF + b < NCH:
            pltpu.make_async_copy(y_hbm.at[pl.ds(0, CH)], rows[b],
                                  ssem[b]).wait()

    plsc.subcore_barrier()
    pltpu.sync_copy(acc.at[pl.ds(s * RPT, RPT)],
                    out_hbm.at[pl.ds(c * NPAD + s * RPT, RPT)])


def _agg(y, src, dst3):
    f = functools.partial(
        pl.kernel,
        out_type=jax.ShapeDtypeStruct((NC * NPAD, H), jnp.float32),
        mesh=_mesh(),
        compiler_params=pltpu.CompilerParams(needs_layout_passes=False),
        scratch_types=(
            [pltpu.VMEM((EPW,), jnp.int32),
             pltpu.VMEM((NCH, CH), jnp.int32)]
            + [pltpu.VMEM((CH, H), jnp.float32) for _ in range(NBUF)]
            + [pltpu.VMEM_SHARED((NPAD, H), jnp.float32)]
            + [pltpu.SemaphoreType.DMA for _ in range(2 * NBUF)]
        ),
    )(_agg_body)
    return f(y, src, dst3).reshape(NC, NPAD, H)


def _lin1_body(x_ref, w_ref, hist_ref, y_ref, dinv_ref):
    deg = 1.0 + jnp.sum(hist_ref[...], axis=1, keepdims=True)
    dinv = lax.rsqrt(deg)
    xw = jnp.dot(x_ref[...], w_ref[...],
                 preferred_element_type=jnp.float32)
    y_ref[...] = xw * dinv
    dinv_ref[...] = dinv


def _lin1(x, W1, hist):
    return pl.pallas_call(
        _lin1_body,
        grid=(NB,),
        in_specs=[
            pl.BlockSpec((R, H), lambda i: (i, 0)),
            pl.BlockSpec((H, H), lambda i: (0, 0)),
            pl.BlockSpec((R, NW), lambda i: (i, 0)),
        ],
        out_specs=[
            pl.BlockSpec((R, H), lambda i: (i, 0)),
            pl.BlockSpec((R, 1), lambda i: (i, 0)),
        ],
        out_shape=[
            jax.ShapeDtypeStruct((N, H), jnp.float32),
            jax.ShapeDtypeStruct((N, 1), jnp.float32),
        ],
    )(x, W1, hist)


def _lin2_body(p0_ref, p1_ref, y_ref, dinv_ref, b_ref, w_ref, y2_ref):
    dinv = dinv_ref[...]
    agg = p0_ref[0] + p1_ref[0] + y_ref[...]
    h = jnp.maximum(agg * dinv + b_ref[...], 0.0)
    y2_ref[...] = jnp.dot(h, w_ref[...],
                          preferred_element_type=jnp.float32) * dinv


def _lin2(parts, y, dinv, b, W2):
    return pl.pallas_call(
        _lin2_body,
        grid=(NB,),
        in_specs=[
            pl.BlockSpec((1, R, H), lambda i: (0, i, 0)),
            pl.BlockSpec((1, R, H), lambda i: (1, i, 0)),
            pl.BlockSpec((R, H), lambda i: (i, 0)),
            pl.BlockSpec((R, 1), lambda i: (i, 0)),
            pl.BlockSpec((1, H), lambda i: (0, 0)),
            pl.BlockSpec((H, H), lambda i: (0, 0)),
        ],
        out_specs=pl.BlockSpec((R, H), lambda i: (i, 0)),
        out_shape=jax.ShapeDtypeStruct((N, H), jnp.float32),
    )(parts, parts, y, dinv, b.reshape(1, H), W2)


def _pool_body(p0_ref, p1_ref, y_ref, dinv_ref, b_ref, bat_ref,
               out_ref, s_sum, s_cnt):
    i = pl.program_id(0)
    agg = p0_ref[0] + p1_ref[0] + y_ref[...]
    h = jnp.maximum(agg * dinv_ref[...] + b_ref[...], 0.0)
    gidx = lax.broadcasted_iota(jnp.int32, (R, G), 1).astype(jnp.float32)
    oh = (bat_ref[...] == gidx).astype(jnp.float32)
    part = lax.dot_general(oh, h, (((0,), (0,)), ((), ())),
                           preferred_element_type=jnp.float32)
    cnt = lax.dot_general(oh, jnp.ones((R, H), jnp.float32),
                          (((0,), (0,)), ((), ())),
                          preferred_element_type=jnp.float32)

    @pl.when(i == 0)
    def _():
        s_sum[...] = part
        s_cnt[...] = cnt

    @pl.when(i > 0)
    def _():
        s_sum[...] += part
        s_cnt[...] += cnt

    @pl.when(i == NB - 1)
    def _():
        out_ref[...] = s_sum[...] / jnp.maximum(s_cnt[...], 1.0)


def _pool(parts, y, dinv, b, bat_col):
    return pl.pallas_call(
        _pool_body,
        grid=(NB,),
        in_specs=[
            pl.BlockSpec((1, R, H), lambda i: (0, i, 0)),
            pl.BlockSpec((1, R, H), lambda i: (1, i, 0)),
            pl.BlockSpec((R, H), lambda i: (i, 0)),
            pl.BlockSpec((R, 1), lambda i: (i, 0)),
            pl.BlockSpec((1, H), lambda i: (0, 0)),
            pl.BlockSpec((R, 1), lambda i: (i, 0)),
        ],
        out_specs=pl.BlockSpec((G, H), lambda i: (0, 0)),
        out_shape=jax.ShapeDtypeStruct((G, H), jnp.float32),
        scratch_shapes=[
            pltpu.VMEM((G, H), jnp.float32),
            pltpu.VMEM((G, H), jnp.float32),
        ],
    )(parts, parts, y, dinv, b.reshape(1, H), bat_col)


@jax.jit
def kernel(x, edge_index, batch, W1, b1, W2, b2):
    src = edge_index[0]
    dst = edge_index[1]
    dst3 = dst.reshape(NW, NCH, CH)
    hist = _deg(dst)
    y1, dinv = _lin1(x, W1, hist.T)
    parts1 = _agg(y1, src, dst3)
    y2 = _lin2(parts1, y1, dinv, b1, W2)
    parts2 = _agg(y2, src, dst3)
    bat_col = batch.astype(jnp.float32).reshape(N, 1)
    return _pool(parts2, y2, dinv, b2, bat_col)


# speedup vs baseline: 1.2969x; 1.2969x over previous
import functools

import jax
import jax.numpy as jnp
from jax import lax
from jax.experimental import pallas as pl
from jax.experimental.pallas import tpu as pltpu
from jax.experimental.pallas import tpu_sc as plsc

N = 10000
E = 320000
D_IN = 128
H = 128
G = 64

NC = 2
NS = 16
NW = NC * NS
EPW = E // NW
CH = 40
NCH = EPW // CH
NPAD = 10240
RPT = NPAD // NS

R = 1000
NB = N // R


def _mesh():
    return plsc.VectorSubcoreMesh(
        core_axis_name="c", subcore_axis_name="s",
        num_cores=NC, num_subcores=NS)


def _deg_body(dst_hbm, out_hbm, idx_v, hist_v):
    c = lax.axis_index("c")
    s = lax.axis_index("s")
    w = c * NS + s
    zero16 = jnp.zeros((16,), jnp.float32)
    one16 = jnp.ones((16,), jnp.float32)

    def zbody(i, carry):
        hist_v[pl.ds(i * 16, 16)] = zero16
        return carry
    lax.fori_loop(0, N // 16, zbody, 0)

    pltpu.sync_copy(dst_hbm.at[pl.ds(w * EPW, EPW)], idx_v)

    def body(i, carry):
        idx = idx_v[pl.ds(i * 16, 16)]
        plsc.addupdate_scatter(hist_v, [idx], one16)
        return carry
    lax.fori_loop(0, EPW // 16, body, 0)

    pltpu.sync_copy(hist_v, out_hbm.at[w])


def _deg(dst):
    f = functools.partial(
        pl.kernel,
        out_type=jax.ShapeDtypeStruct((NW, N), jnp.float32),
        mesh=_mesh(),
        compiler_params=pltpu.CompilerParams(needs_layout_passes=False),
        scratch_types=[
            pltpu.VMEM((EPW,), jnp.int32),
            pltpu.VMEM((N,), jnp.float32),
        ],
    )(_deg_body)
    return f(dst)


NBUF = 7
NOUT = (NCH + NBUF - 1) // NBUF


def _agg_body(y_hbm, src_hbm, dst_hbm, out_hbm, sidx,
              d0, d1, d2, d3, d4, d5, d6,
              r0, r1, r2, r3, r4, r5, r6, acc,
              i0, i1, i2, i3, i4, i5, i6,
              g0, g1, g2, g3, g4, g5, g6,
              s0, s1, s2, s3, s4, s5, s6):
    c = lax.axis_index("c")
    s = lax.axis_index("s")
    w = c * NS + s
    dbuf = (d0, d1, d2, d3, d4, d5, d6)
    rows = (r0, r1, r2, r3, r4, r5, r6)
    isem = (i0, i1, i2, i3, i4, i5, i6)
    gsem = (g0, g1, g2, g3, g4, g5, g6)
    ssem = (s0, s1, s2, s3, s4, s5, s6)
    zero16 = jnp.zeros((16,), jnp.float32)

    def zr(i, carry):
        def zc(j, carry2):
            r0[i, pl.ds(j * 16, 16)] = zero16
            return carry2
        return lax.fori_loop(0, H // 16, zc, carry)
    lax.fori_loop(0, CH, zr, 0)

    def za(k, carry):
        pltpu.sync_copy(r0, acc.at[pl.ds(s * RPT + k * CH, CH)])
        return carry
    lax.fori_loop(0, RPT // CH, za, 0)

    pltpu.sync_copy(src_hbm.at[pl.ds(w * EPW, EPW)], sidx)
    plsc.subcore_barrier()

    def outer(g, carry):
        for b in range(NBUF):
            i = g * NBUF + b

            @pl.when(g > 0)
            def _(b=b):
                pltpu.make_async_copy(y_hbm.at[pl.ds(0, CH)], rows[b],
                                      ssem[b]).wait()

            @pl.when(i < NCH)
            def _(b=b, i=i):
                base = w * EPW + i * CH
                pltpu.async_copy(dst_hbm.at[pl.ds(base, CH)], dbuf[b],
                                 isem[b])
                pltpu.async_copy(y_hbm.at[sidx.at[pl.ds(i * CH, CH)]],
                                 rows[b], gsem[b])
        for b in range(NBUF):
            i = g * NBUF + b

            @pl.when(i < NCH)
            def _(b=b, i=i):
                pltpu.make_async_copy(y_hbm.at[pl.ds(0, CH)], rows[b],
                                      gsem[b]).wait()
                pltpu.make_async_copy(dst_hbm.at[pl.ds(0, CH)], dbuf[b],
                                      isem[b]).wait()
                pltpu.async_copy(rows[b], acc.at[dbuf[b]], ssem[b],
                                 add=True)
        return carry
    lax.fori_loop(0, NOUT, outer, 0)

    for b in range(NBUF):
        if (NOUT - 1) * NBUF + b < NCH:
            pltpu.make_async_copy(y_hbm.at[pl.ds(0, CH)], rows[b],
                                  ssem[b]).wait()

    plsc.subcore_barrier()
    pltpu.sync_copy(acc.at[pl.ds(s * RPT, RPT)],
                    out_hbm.at[pl.ds(c * NPAD + s * RPT, RPT)])


def _agg(y, src, dst):
    f = functools.partial(
        pl.kernel,
        out_type=jax.ShapeDtypeStruct((NC * NPAD, H), jnp.float32),
        mesh=_mesh(),
        compiler_params=pltpu.CompilerParams(needs_layout_passes=False),
        scratch_types=(
            [pltpu.VMEM((EPW,), jnp.int32)]
            + [pltpu.VMEM((CH,), jnp.int32) for _ in range(NBUF)]
            + [pltpu.VMEM((CH, H), jnp.float32) for _ in range(NBUF)]
            + [pltpu.VMEM_SHARED((NPAD, H), jnp.float32)]
            + [pltpu.SemaphoreType.DMA for _ in range(3 * NBUF)]
        ),
    )(_agg_body)
    return f(y, src, dst).reshape(NC, NPAD, H)


def _lin1_body(x_ref, w_ref, hist_ref, y_ref, dinv_ref):
    deg = 1.0 + jnp.sum(hist_ref[...], axis=1, keepdims=True)
    dinv = lax.rsqrt(deg)
    xw = jnp.dot(x_ref[...], w_ref[...],
                 preferred_element_type=jnp.float32)
    y_ref[...] = xw * dinv
    dinv_ref[...] = dinv


def _lin1(x, W1, hist):
    return pl.pallas_call(
        _lin1_body,
        grid=(NB,),
        in_specs=[
            pl.BlockSpec((R, H), lambda i: (i, 0)),
            pl.BlockSpec((H, H), lambda i: (0, 0)),
            pl.BlockSpec((R, NW), lambda i: (i, 0)),
        ],
        out_specs=[
            pl.BlockSpec((R, H), lambda i: (i, 0)),
            pl.BlockSpec((R, 1), lambda i: (i, 0)),
        ],
        out_shape=[
            jax.ShapeDtypeStruct((N, H), jnp.float32),
            jax.ShapeDtypeStruct((N, 1), jnp.float32),
        ],
    )(x, W1, hist)


def _lin2_body(p0_ref, p1_ref, y_ref, dinv_ref, b_ref, w_ref, y2_ref):
    dinv = dinv_ref[...]
    agg = p0_ref[0] + p1_ref[0] + y_ref[...]
    h = jnp.maximum(agg * dinv + b_ref[...], 0.0)
    y2_ref[...] = jnp.dot(h, w_ref[...],
                          preferred_element_type=jnp.float32) * dinv


def _lin2(parts, y, dinv, b, W2):
    return pl.pallas_call(
        _lin2_body,
        grid=(NB,),
        in_specs=[
            pl.BlockSpec((1, R, H), lambda i: (0, i, 0)),
            pl.BlockSpec((1, R, H), lambda i: (1, i, 0)),
            pl.BlockSpec((R, H), lambda i: (i, 0)),
            pl.BlockSpec((R, 1), lambda i: (i, 0)),
            pl.BlockSpec((1, H), lambda i: (0, 0)),
            pl.BlockSpec((H, H), lambda i: (0, 0)),
        ],
        out_specs=pl.BlockSpec((R, H), lambda i: (i, 0)),
        out_shape=jax.ShapeDtypeStruct((N, H), jnp.float32),
    )(parts, parts, y, dinv, b.reshape(1, H), W2)


def _pool_body(p0_ref, p1_ref, y_ref, dinv_ref, b_ref, bat_ref,
               out_ref, s_sum, s_cnt):
    i = pl.program_id(0)
    agg = p0_ref[0] + p1_ref[0] + y_ref[...]
    h = jnp.maximum(agg * dinv_ref[...] + b_ref[...], 0.0)
    gidx = lax.broadcasted_iota(jnp.int32, (R, G), 1).astype(jnp.float32)
    oh = (bat_ref[...] == gidx).astype(jnp.float32)
    part = lax.dot_general(oh, h, (((0,), (0,)), ((), ())),
                           preferred_element_type=jnp.float32)
    cnt = lax.dot_general(oh, jnp.ones((R, H), jnp.float32),
                          (((0,), (0,)), ((), ())),
                          preferred_element_type=jnp.float32)

    @pl.when(i == 0)
    def _():
        s_sum[...] = part
        s_cnt[...] = cnt

    @pl.when(i > 0)
    def _():
        s_sum[...] += part
        s_cnt[...] += cnt

    @pl.when(i == NB - 1)
    def _():
        out_ref[...] = s_sum[...] / jnp.maximum(s_cnt[...], 1.0)


def _pool(parts, y, dinv, b, bat_col):
    return pl.pallas_call(
        _pool_body,
        grid=(NB,),
        in_specs=[
            pl.BlockSpec((1, R, H), lambda i: (0, i, 0)),
            pl.BlockSpec((1, R, H), lambda i: (1, i, 0)),
            pl.BlockSpec((R, H), lambda i: (i, 0)),
            pl.BlockSpec((R, 1), lambda i: (i, 0)),
            pl.BlockSpec((1, H), lambda i: (0, 0)),
            pl.BlockSpec((R, 1), lambda i: (i, 0)),
        ],
        out_specs=pl.BlockSpec((G, H), lambda i: (0, 0)),
        out_shape=jax.ShapeDtypeStruct((G, H), jnp.float32),
        scratch_shapes=[
            pltpu.VMEM((G, H), jnp.float32),
            pltpu.VMEM((G, H), jnp.float32),
        ],
    )(parts, parts, y, dinv, b.reshape(1, H), bat_col)


@jax.jit
def kernel(x, edge_index, batch, W1, b1, W2, b2):
    src = edge_index[0]
    dst = edge_index[1]
    hist = _deg(dst)
    y1, dinv = _lin1(x, W1, hist.T)
    parts1 = _agg(y1, src, dst)
    y2 = _lin2(parts1, y1, dinv, b1, W2)
    parts2 = _agg(y2, src, dst)
    bat_col = batch.astype(jnp.float32).reshape(N, 1)
    return _pool(parts2, y2, dinv, b2, bat_col)


# speedup vs baseline: 1.3279x; 1.0238x over previous
import functools

import jax
import jax.numpy as jnp
from jax import lax
from jax.experimental import pallas as pl
from jax.experimental.pallas import tpu as pltpu
from jax.experimental.pallas import tpu_sc as plsc

N = 10000
E = 320000
D_IN = 128
H = 128
G = 64

NC = 2
NS = 16
NW = NC * NS
EPW = E // NW
CH = 40
NCH = EPW // CH
NPAD = 10240
RPT = NPAD // NS

R = 2000
NB = N // R


def _mesh():
    return plsc.VectorSubcoreMesh(
        core_axis_name="c", subcore_axis_name="s",
        num_cores=NC, num_subcores=NS)


def _deg_body(dst_hbm, out_hbm, idx_v, hist_v):
    c = lax.axis_index("c")
    s = lax.axis_index("s")
    w = c * NS + s
    zero16 = jnp.zeros((16,), jnp.float32)
    one16 = jnp.ones((16,), jnp.float32)

    def zbody(i, carry):
        hist_v[pl.ds(i * 16, 16)] = zero16
        return carry
    lax.fori_loop(0, N // 16, zbody, 0)

    pltpu.sync_copy(dst_hbm.at[pl.ds(w * EPW, EPW)], idx_v)

    def body(i, carry):
        idx = idx_v[pl.ds(i * 16, 16)]
        plsc.addupdate_scatter(hist_v, [idx], one16)
        return carry
    lax.fori_loop(0, EPW // 16, body, 0)

    pltpu.sync_copy(hist_v, out_hbm.at[w])


def _deg(dst):
    f = functools.partial(
        pl.kernel,
        out_type=jax.ShapeDtypeStruct((NW, N), jnp.float32),
        mesh=_mesh(),
        compiler_params=pltpu.CompilerParams(needs_layout_passes=False),
        scratch_types=[
            pltpu.VMEM((EPW,), jnp.int32),
            pltpu.VMEM((N,), jnp.float32),
        ],
    )(_deg_body)
    return f(dst)


NBUF = 7
NOUT = (NCH + NBUF - 1) // NBUF


def _agg_body(y_hbm, src_hbm, dst_hbm, out_hbm, sidx,
              d0, d1, d2, d3, d4, d5, d6,
              r0, r1, r2, r3, r4, r5, r6, acc,
              i0, i1, i2, i3, i4, i5, i6,
              g0, g1, g2, g3, g4, g5, g6,
              s0, s1, s2, s3, s4, s5, s6):
    c = lax.axis_index("c")
    s = lax.axis_index("s")
    w = c * NS + s
    dbuf = (d0, d1, d2, d3, d4, d5, d6)
    rows = (r0, r1, r2, r3, r4, r5, r6)
    isem = (i0, i1, i2, i3, i4, i5, i6)
    gsem = (g0, g1, g2, g3, g4, g5, g6)
    ssem = (s0, s1, s2, s3, s4, s5, s6)
    zero16 = jnp.zeros((16,), jnp.float32)

    def zr(i, carry):
        def zc(j, carry2):
            r0[i, pl.ds(j * 16, 16)] = zero16
            return carry2
        return lax.fori_loop(0, H // 16, zc, carry)
    lax.fori_loop(0, CH, zr, 0)

    def za(k, carry):
        pltpu.sync_copy(r0, acc.at[pl.ds(s * RPT + k * CH, CH)])
        return carry
    lax.fori_loop(0, RPT // CH, za, 0)

    pltpu.sync_copy(src_hbm.at[pl.ds(w * EPW, EPW)], sidx)
    plsc.subcore_barrier()

    def outer(g, carry):
        for b in range(NBUF):
            i = g * NBUF + b

            @pl.when(g > 0)
            def _(b=b):
                pltpu.make_async_copy(y_hbm.at[pl.ds(0, CH)], rows[b],
                                      ssem[b]).wait()

            @pl.when(i < NCH)
            def _(b=b, i=i):
                base = w * EPW + i * CH
                pltpu.async_copy(dst_hbm.at[pl.ds(base, CH)], dbuf[b],
                                 isem[b])
                pltpu.async_copy(y_hbm.at[sidx.at[pl.ds(i * CH, CH)]],
                                 rows[b], gsem[b])
        for b in range(NBUF):
            i = g * NBUF + b

            @pl.when(i < NCH)
            def _(b=b, i=i):
                pltpu.make_async_copy(y_hbm.at[pl.ds(0, CH)], rows[b],
                                      gsem[b]).wait()
                pltpu.make_async_copy(dst_hbm.at[pl.ds(0, CH)], dbuf[b],
                                      isem[b]).wait()
                pltpu.async_copy(rows[b], acc.at[dbuf[b]], ssem[b],
                                 add=True)
        return carry
    lax.fori_loop(0, NOUT, outer, 0)

    for b in range(NBUF):
        if (NOUT - 1) * NBUF + b < NCH:
            pltpu.make_async_copy(y_hbm.at[pl.ds(0, CH)], rows[b],
                                  ssem[b]).wait()

    plsc.subcore_barrier()
    pltpu.sync_copy(acc.at[pl.ds(s * RPT, RPT)],
                    out_hbm.at[pl.ds(c * NPAD + s * RPT, RPT)])


def _agg(y, src, dst):
    f = functools.partial(
        pl.kernel,
        out_type=jax.ShapeDtypeStruct((NC * NPAD, H), jnp.float32),
        mesh=_mesh(),
        compiler_params=pltpu.CompilerParams(needs_layout_passes=False),
        scratch_types=(
            [pltpu.VMEM((EPW,), jnp.int32)]
            + [pltpu.VMEM((CH,), jnp.int32) for _ in range(NBUF)]
            + [pltpu.VMEM((CH, H), jnp.float32) for _ in range(NBUF)]
            + [pltpu.VMEM_SHARED((NPAD, H), jnp.float32)]
            + [pltpu.SemaphoreType.DMA for _ in range(3 * NBUF)]
        ),
    )(_agg_body)
    return f(y, src, dst).reshape(NC, NPAD, H)


def _lin1_body(x_ref, w_ref, hist_ref, y_ref, dinv_ref):
    deg = 1.0 + jnp.sum(hist_ref[...], axis=1, keepdims=True)
    dinv = lax.rsqrt(deg)
    xw = jnp.dot(x_ref[...], w_ref[...],
                 preferred_element_type=jnp.float32)
    y_ref[...] = xw * dinv
    dinv_ref[...] = dinv


def _lin1(x, W1, hist):
    return pl.pallas_call(
        _lin1_body,
        grid=(NB,),
        in_specs=[
            pl.BlockSpec((R, H), lambda i: (i, 0)),
            pl.BlockSpec((H, H), lambda i: (0, 0)),
            pl.BlockSpec((R, NW), lambda i: (i, 0)),
        ],
        out_specs=[
            pl.BlockSpec((R, H), lambda i: (i, 0)),
            pl.BlockSpec((R, 1), lambda i: (i, 0)),
        ],
        out_shape=[
            jax.ShapeDtypeStruct((N, H), jnp.float32),
            jax.ShapeDtypeStruct((N, 1), jnp.float32),
        ],
    )(x, W1, hist)


def _lin2_body(p0_ref, p1_ref, y_ref, dinv_ref, b_ref, w_ref, y2_ref):
    dinv = dinv_ref[...]
    agg = p0_ref[0] + p1_ref[0] + y_ref[...]
    h = jnp.maximum(agg * dinv + b_ref[...], 0.0)
    y2_ref[...] = jnp.dot(h, w_ref[...],
                          preferred_element_type=jnp.float32) * dinv


def _lin2(parts, y, dinv, b, W2):
    return pl.pallas_call(
        _lin2_body,
        grid=(NB,),
        in_specs=[
            pl.BlockSpec((1, R, H), lambda i: (0, i, 0)),
            pl.BlockSpec((1, R, H), lambda i: (1, i, 0)),
            pl.BlockSpec((R, H), lambda i: (i, 0)),
            pl.BlockSpec((R, 1), lambda i: (i, 0)),
            pl.BlockSpec((1, H), lambda i: (0, 0)),
            pl.BlockSpec((H, H), lambda i: (0, 0)),
        ],
        out_specs=pl.BlockSpec((R, H), lambda i: (i, 0)),
        out_shape=jax.ShapeDtypeStruct((N, H), jnp.float32),
    )(parts, parts, y, dinv, b.reshape(1, H), W2)


def _pool_body(p0_ref, p1_ref, y_ref, dinv_ref, b_ref, bat_ref,
               out_ref, s_sum, s_cnt):
    i = pl.program_id(0)
    agg = p0_ref[0] + p1_ref[0] + y_ref[...]
    h = jnp.maximum(agg * dinv_ref[...] + b_ref[...], 0.0)
    gidx = lax.broadcasted_iota(jnp.int32, (R, G), 1).astype(jnp.float32)
    oh = (bat_ref[...] == gidx).astype(jnp.float32)
    part = lax.dot_general(oh, h, (((0,), (0,)), ((), ())),
                           preferred_element_type=jnp.float32)
    cnt = lax.dot_general(oh, jnp.ones((R, H), jnp.float32),
                          (((0,), (0,)), ((), ())),
                          preferred_element_type=jnp.float32)

    @pl.when(i == 0)
    def _():
        s_sum[...] = part
        s_cnt[...] = cnt

    @pl.when(i > 0)
    def _():
        s_sum[...] += part
        s_cnt[...] += cnt

    @pl.when(i == NB - 1)
    def _():
        out_ref[...] = s_sum[...] / jnp.maximum(s_cnt[...], 1.0)


def _pool(parts, y, dinv, b, bat_col):
    return pl.pallas_call(
        _pool_body,
        grid=(NB,),
        in_specs=[
            pl.BlockSpec((1, R, H), lambda i: (0, i, 0)),
            pl.BlockSpec((1, R, H), lambda i: (1, i, 0)),
            pl.BlockSpec((R, H), lambda i: (i, 0)),
            pl.BlockSpec((R, 1), lambda i: (i, 0)),
            pl.BlockSpec((1, H), lambda i: (0, 0)),
            pl.BlockSpec((R, 1), lambda i: (i, 0)),
        ],
        out_specs=pl.BlockSpec((G, H), lambda i: (0, 0)),
        out_shape=jax.ShapeDtypeStruct((G, H), jnp.float32),
        scratch_shapes=[
            pltpu.VMEM((G, H), jnp.float32),
            pltpu.VMEM((G, H), jnp.float32),
        ],
    )(parts, parts, y, dinv, b.reshape(1, H), bat_col)


@jax.jit
def kernel(x, edge_index, batch, W1, b1, W2, b2):
    src = edge_index[0]
    dst = edge_index[1]
    hist = _deg(dst)
    y1, dinv = _lin1(x, W1, hist.T)
    parts1 = _agg(y1, src, dst)
    y2 = _lin2(parts1, y1, dinv, b1, W2)
    parts2 = _agg(y2, src, dst)
    bat_col = batch.astype(jnp.float32).reshape(N, 1)
    return _pool(parts2, y2, dinv, b2, bat_col)


# speedup vs baseline: 1.3477x; 1.0149x over previous
import functools

import jax
import jax.numpy as jnp
from jax import lax
from jax.experimental import pallas as pl
from jax.experimental.pallas import tpu as pltpu
from jax.experimental.pallas import tpu_sc as plsc

N = 10000
E = 320000
D_IN = 128
H = 128
G = 64

NC = 2
NS = 16
NW = NC * NS
EPW = E // NW
CH = 40
NCH = EPW // CH
NPAD = 10240
RPT = NPAD // NS

R = 2000
NB = N // R


def _mesh():
    return plsc.VectorSubcoreMesh(
        core_axis_name="c", subcore_axis_name="s",
        num_cores=NC, num_subcores=NS)


def _deg_body(dst_hbm, out_hbm, idx_v, hist_v):
    c = lax.axis_index("c")
    s = lax.axis_index("s")
    w = c * NS + s
    zero16 = jnp.zeros((16,), jnp.float32)
    one16 = jnp.ones((16,), jnp.float32)

    def zbody(i, carry):
        hist_v[pl.ds(i * 16, 16)] = zero16
        return carry
    lax.fori_loop(0, N // 16, zbody, 0)

    pltpu.sync_copy(dst_hbm.at[pl.ds(w * EPW, EPW)], idx_v)

    def body(i, carry):
        idx = idx_v[pl.ds(i * 16, 16)]
        plsc.addupdate_scatter(hist_v, [idx], one16)
        return carry
    lax.fori_loop(0, EPW // 16, body, 0)

    pltpu.sync_copy(hist_v, out_hbm.at[w])


def _deg(dst):
    f = functools.partial(
        pl.kernel,
        out_type=jax.ShapeDtypeStruct((NW, N), jnp.float32),
        mesh=_mesh(),
        compiler_params=pltpu.CompilerParams(needs_layout_passes=False),
        scratch_types=[
            pltpu.VMEM((EPW,), jnp.int32),
            pltpu.VMEM((N,), jnp.float32),
        ],
    )(_deg_body)
    return f(dst)


NBUF = 7
NOUT = (NCH + NBUF - 1) // NBUF


def _agg_body(y_hbm, src_hbm, dst_hbm, out_hbm, sidx,
              d0, d1, d2, d3, d4, d5, d6,
              r0, r1, r2, r3, r4, r5, r6, acc,
              i0, i1, i2, i3, i4, i5, i6,
              g0, g1, g2, g3, g4, g5, g6,
              s0, s1, s2, s3, s4, s5, s6):
    c = lax.axis_index("c")
    s = lax.axis_index("s")
    w = c * NS + s
    dbuf = (d0, d1, d2, d3, d4, d5, d6)
    rows = (r0, r1, r2, r3, r4, r5, r6)
    isem = (i0, i1, i2, i3, i4, i5, i6)
    gsem = (g0, g1, g2, g3, g4, g5, g6)
    ssem = (s0, s1, s2, s3, s4, s5, s6)
    zero16 = jnp.zeros((16,), jnp.float32)

    def zr(i, carry):
        def zc(j, carry2):
            r0[i, pl.ds(j * 16, 16)] = zero16
            return carry2
        return lax.fori_loop(0, H // 16, zc, carry)
    lax.fori_loop(0, CH, zr, 0)

    sidx_d = pltpu.async_copy(src_hbm.at[pl.ds(w * EPW, EPW)], sidx, i0)
    zdescs = [pltpu.async_copy(r0, acc.at[pl.ds(s * RPT + k * CH, CH)], g0)
              for k in range(RPT // CH)]
    for d in zdescs:
        d.wait()
    sidx_d.wait()
    plsc.subcore_barrier()

    def outer(g, carry):
        for b in range(NBUF):
            i = g * NBUF + b

            @pl.when(g > 0)
            def _(b=b):
                pltpu.make_async_copy(y_hbm.at[pl.ds(0, CH)], rows[b],
                                      ssem[b]).wait()

            @pl.when(i < NCH)
            def _(b=b, i=i):
                base = w * EPW + i * CH
                pltpu.async_copy(dst_hbm.at[pl.ds(base, CH)], dbuf[b],
                                 isem[b])
                pltpu.async_copy(y_hbm.at[sidx.at[pl.ds(i * CH, CH)]],
                                 rows[b], gsem[b])
        for b in range(NBUF):
            i = g * NBUF + b

            @pl.when(i < NCH)
            def _(b=b, i=i):
                pltpu.make_async_copy(y_hbm.at[pl.ds(0, CH)], rows[b],
                                      gsem[b]).wait()
                pltpu.make_async_copy(dst_hbm.at[pl.ds(0, CH)], dbuf[b],
                                      isem[b]).wait()
                pltpu.async_copy(rows[b], acc.at[dbuf[b]], ssem[b],
                                 add=True)
        return carry
    lax.fori_loop(0, NOUT, outer, 0)

    for b in range(NBUF):
        if (NOUT - 1) * NBUF + b < NCH:
            pltpu.make_async_copy(y_hbm.at[pl.ds(0, CH)], rows[b],
                                  ssem[b]).wait()

    plsc.subcore_barrier()
    pltpu.sync_copy(acc.at[pl.ds(s * RPT, RPT)],
                    out_hbm.at[pl.ds(c * NPAD + s * RPT, RPT)])


def _agg(y, src, dst):
    f = functools.partial(
        pl.kernel,
        out_type=jax.ShapeDtypeStruct((NC * NPAD, H), jnp.float32),
        mesh=_mesh(),
        compiler_params=pltpu.CompilerParams(needs_layout_passes=False),
        scratch_types=(
            [pltpu.VMEM((EPW,), jnp.int32)]
            + [pltpu.VMEM((CH,), jnp.int32) for _ in range(NBUF)]
            + [pltpu.VMEM((CH, H), jnp.float32) for _ in range(NBUF)]
            + [pltpu.VMEM_SHARED((NPAD, H), jnp.float32)]
            + [pltpu.SemaphoreType.DMA for _ in range(3 * NBUF)]
        ),
    )(_agg_body)
    return f(y, src, dst).reshape(NC, NPAD, H)


def _lin1_body(x_ref, w_ref, hist_ref, y_ref, dinv_ref):
    deg = 1.0 + jnp.sum(hist_ref[...], axis=1, keepdims=True)
    dinv = lax.rsqrt(deg)
    xw = jnp.dot(x_ref[...], w_ref[...],
                 preferred_element_type=jnp.float32)
    y_ref[...] = xw * dinv
    dinv_ref[...] = dinv


def _lin1(x, W1, hist):
    return pl.pallas_call(
        _lin1_body,
        grid=(NB,),
        in_specs=[
            pl.BlockSpec((R, H), lambda i: (i, 0)),
            pl.BlockSpec((H, H), lambda i: (0, 0)),
            pl.BlockSpec((R, NW), lambda i: (i, 0)),
        ],
        out_specs=[
            pl.BlockSpec((R, H), lambda i: (i, 0)),
            pl.BlockSpec((R, 1), lambda i: (i, 0)),
        ],
        out_shape=[
            jax.ShapeDtypeStruct((N, H), jnp.float32),
            jax.ShapeDtypeStruct((N, 1), jnp.float32),
        ],
    )(x, W1, hist)


def _lin2_body(p0_ref, p1_ref, y_ref, dinv_ref, b_ref, w_ref, y2_ref):
    dinv = dinv_ref[...]
    agg = p0_ref[0] + p1_ref[0] + y_ref[...]
    h = jnp.maximum(agg * dinv + b_ref[...], 0.0)
    y2_ref[...] = jnp.dot(h, w_ref[...],
                          preferred_element_type=jnp.float32) * dinv


def _lin2(parts, y, dinv, b, W2):
    return pl.pallas_call(
        _lin2_body,
        grid=(NB,),
        in_specs=[
            pl.BlockSpec((1, R, H), lambda i: (0, i, 0)),
            pl.BlockSpec((1, R, H), lambda i: (1, i, 0)),
            pl.BlockSpec((R, H), lambda i: (i, 0)),
            pl.BlockSpec((R, 1), lambda i: (i, 0)),
            pl.BlockSpec((1, H), lambda i: (0, 0)),
            pl.BlockSpec((H, H), lambda i: (0, 0)),
        ],
        out_specs=pl.BlockSpec((R, H), lambda i: (i, 0)),
        out_shape=jax.ShapeDtypeStruct((N, H), jnp.float32),
    )(parts, parts, y, dinv, b.reshape(1, H), W2)


def _pool_body(p0_ref, p1_ref, y_ref, dinv_ref, b_ref, bat_ref,
               out_ref, s_sum, s_cnt):
    i = pl.program_id(0)
    agg = p0_ref[0] + p1_ref[0] + y_ref[...]
    h = jnp.maximum(agg * dinv_ref[...] + b_ref[...], 0.0)
    gidx = lax.broadcasted_iota(jnp.int32, (R, G), 1).astype(jnp.float32)
    oh = (bat_ref[...] == gidx).astype(jnp.float32)
    part = lax.dot_general(oh, h, (((0,), (0,)), ((), ())),
                           preferred_element_type=jnp.float32)
    cnt = lax.dot_general(oh, jnp.ones((R, H), jnp.float32),
                          (((0,), (0,)), ((), ())),
                          preferred_element_type=jnp.float32)

    @pl.when(i == 0)
    def _():
        s_sum[...] = part
        s_cnt[...] = cnt

    @pl.when(i > 0)
    def _():
        s_sum[...] += part
        s_cnt[...] += cnt

    @pl.when(i == NB - 1)
    def _():
        out_ref[...] = s_sum[...] / jnp.maximum(s_cnt[...], 1.0)


def _pool(parts, y, dinv, b, bat_col):
    return pl.pallas_call(
        _pool_body,
        grid=(NB,),
        in_specs=[
            pl.BlockSpec((1, R, H), lambda i: (0, i, 0)),
            pl.BlockSpec((1, R, H), lambda i: (1, i, 0)),
            pl.BlockSpec((R, H), lambda i: (i, 0)),
            pl.BlockSpec((R, 1), lambda i: (i, 0)),
            pl.BlockSpec((1, H), lambda i: (0, 0)),
            pl.BlockSpec((R, 1), lambda i: (i, 0)),
        ],
        out_specs=pl.BlockSpec((G, H), lambda i: (0, 0)),
        out_shape=jax.ShapeDtypeStruct((G, H), jnp.float32),
        scratch_shapes=[
            pltpu.VMEM((G, H), jnp.float32),
            pltpu.VMEM((G, H), jnp.float32),
        ],
    )(parts, parts, y, dinv, b.reshape(1, H), bat_col)


@jax.jit
def kernel(x, edge_index, batch, W1, b1, W2, b2):
    src = edge_index[0]
    dst = edge_index[1]
    hist = _deg(dst)
    y1, dinv = _lin1(x, W1, hist.T)
    parts1 = _agg(y1, src, dst)
    y2 = _lin2(parts1, y1, dinv, b1, W2)
    parts2 = _agg(y2, src, dst)
    bat_col = batch.astype(jnp.float32).reshape(N, 1)
    return _pool(parts2, y2, dinv, b2, bat_col)
